# baseline (speedup 1.0000x reference)
import functools

import jax
import jax.numpy as jnp
from jax import lax
from jax.experimental import pallas as pl
from jax.experimental.pallas import tpu as pltpu
from jax.experimental.pallas import tpu_sc as plsc

NNODE = 10000
NEDGE = 320000
DIM = 128
NGRAPH = 64
NCLS = 2

NCORE = 2
NSUB = 16
NWORK = NCORE * NSUB
BLK = 64
ETOT = NEDGE + NNODE
CHKB = 24
NBLK = -(-(-(-ETOT // (NWORK * BLK))) // CHKB) * CHKB
NCHK = NBLK // CHKB
EPAD = NWORK * NBLK * BLK
NPAD = 10112
ROWS_PER_TILE = NPAD // NSUB
DEGW = 16
DEPTH = 4
AHEAD = 2

_mesh = plsc.VectorSubcoreMesh(core_axis_name="c", subcore_axis_name="s")


@functools.partial(
    pl.kernel,
    out_type=jax.ShapeDtypeStruct((NCORE, NPAD, DEGW), jnp.float32),
    mesh=_mesh,
    scratch_types=[
        pltpu.VMEM((CHKB, BLK), jnp.int32),
        pltpu.VMEM((BLK, DEGW), jnp.float32),
        pltpu.VMEM((8, DEGW), jnp.float32),
        pltpu.VMEM_SHARED((NPAD, DEGW), jnp.float32),
    ],
)
def _sc_degree(dst_hbm, out_hbm, didx, ones_buf, zbuf, accum):
    cid = lax.axis_index("c")
    sid = lax.axis_index("s")
    wid = cid * NSUB + sid

    @pl.loop(0, BLK)
    def _(i):
        ones_buf[i, pl.ds(0, DEGW)] = jnp.ones((DEGW,), jnp.float32)

    @pl.loop(0, 8)
    def _(i):
        zbuf[i, pl.ds(0, DEGW)] = jnp.zeros((DEGW,), jnp.float32)

    base = sid * ROWS_PER_TILE

    @pl.loop(0, ROWS_PER_TILE // 8)
    def _(r):
        pltpu.sync_copy(zbuf, accum.at[pl.ds(base + r * 8, 8)])

    plsc.subcore_barrier()

    @pl.loop(0, NCHK)
    def _(c):
        pltpu.sync_copy(dst_hbm.at[wid, pl.ds(c * CHKB, CHKB)], didx)

        @pl.loop(0, CHKB)
        def _(j):
            pltpu.sync_copy(ones_buf, accum.at[didx.at[j]], add=True)

    plsc.subcore_barrier()
    pltpu.sync_copy(accum.at[pl.ds(base, ROWS_PER_TILE)],
                    out_hbm.at[cid, pl.ds(base, ROWS_PER_TILE)])


@functools.partial(
    pl.kernel,
    out_type=jax.ShapeDtypeStruct((NCORE, NPAD, DIM), jnp.float32),
    mesh=_mesh,
    scratch_types=[
        pltpu.VMEM((CHKB, BLK), jnp.int32),
        pltpu.VMEM((CHKB, BLK), jnp.int32),
        pltpu.VMEM((BLK, DIM), jnp.float32),
        pltpu.VMEM((BLK, DIM), jnp.float32),
        pltpu.VMEM((BLK, DIM), jnp.float32),
        pltpu.VMEM((BLK, DIM), jnp.float32),
        pltpu.VMEM_SHARED((NPAD, DIM), jnp.float32),
        pltpu.SemaphoreType.DMA,
        pltpu.SemaphoreType.DMA,
        pltpu.SemaphoreType.DMA,
        pltpu.SemaphoreType.DMA,
        pltpu.SemaphoreType.DMA,
        pltpu.SemaphoreType.DMA,
        pltpu.SemaphoreType.DMA,
        pltpu.SemaphoreType.DMA,
    ],
)
def _sc_propagate(z_hbm, src_hbm, dst_hbm, out_hbm,
                  sidx, didx, buf0, buf1, buf2, buf3, accum,
                  gs0, gs1, gs2, gs3, ss0, ss1, ss2, ss3):
    cid = lax.axis_index("c")
    sid = lax.axis_index("s")
    wid = cid * NSUB + sid
    bufs = (buf0, buf1, buf2, buf3)
    gsem = (gs0, gs1, gs2, gs3)
    ssem = (ss0, ss1, ss2, ss3)

    @pl.loop(0, 8)
    def _(i):
        for k in range(DIM // 16):
            buf0[i, pl.ds(k * 16, 16)] = jnp.zeros((16,), jnp.float32)

    base = sid * ROWS_PER_TILE

    @pl.loop(0, ROWS_PER_TILE // 8)
    def _(r):
        pltpu.sync_copy(buf0.at[pl.ds(0, 8)], accum.at[pl.ds(base + r * 8, 8)])

    plsc.subcore_barrier()

    @pl.loop(0, NCHK)
    def _(c):
        @pl.when(c > 0)
        def _():
            for b in range(DEPTH):
                pltpu.make_async_copy(
                    bufs[b], accum.at[didx.at[CHKB - DEPTH + b]],
                    ssem[b]).wait()

        pltpu.sync_copy(src_hbm.at[wid, pl.ds(c * CHKB, CHKB)], sidx)
        pltpu.sync_copy(dst_hbm.at[wid, pl.ds(c * CHKB, CHKB)], didx)

        for jg in range(AHEAD):
            pltpu.async_copy(z_hbm.at[sidx.at[jg]], bufs[jg], gsem[jg])

        @pl.loop(0, CHKB, step=DEPTH)
        def _(j0):
            for b in range(DEPTH):
                j = j0 + b
                jg = j + AHEAD
                bg = (b + AHEAD) % DEPTH

                @pl.when(jg < CHKB)
                def _():
                    @pl.when(jg >= DEPTH)
                    def _():
                        pltpu.make_async_copy(
                            bufs[bg], accum.at[didx.at[jg - DEPTH]],
                            ssem[bg]).wait()

                    pltpu.async_copy(z_hbm.at[sidx.at[jg]], bufs[bg],
                                     gsem[bg])

                pltpu.make_async_copy(z_hbm.at[sidx.at[j]], bufs[b],
                                      gsem[b]).wait()
                pltpu.async_copy(bufs[b], accum.at[didx.at[j]], ssem[b],
                                 add=True)

    for b in range(DEPTH):
        pltpu.make_async_copy(bufs[b], accum.at[didx.at[CHKB - DEPTH + b]],
                              ssem[b]).wait()

    plsc.subcore_barrier()
    pltpu.sync_copy(accum.at[pl.ds(base, ROWS_PER_TILE)],
                    out_hbm.at[cid, pl.ds(base, ROWS_PER_TILE)])


def _tc_first_body(x_ref, w_ref, degp_ref, z_ref, dinv_ref):
    degp = degp_ref[...]
    deg = degp[0, :NNODE, 0:1] + degp[1, :NNODE, 0:1]
    dinv = lax.rsqrt(deg)
    z = jnp.dot(x_ref[...], w_ref[...], preferred_element_type=jnp.float32)
    z_ref[...] = z * dinv
    dinv_ref[...] = dinv


def _tc_mid_body(sp_ref, dinv_ref, b_ref, w_ref, z_ref):
    s = sp_ref[0, :NNODE, :] + sp_ref[1, :NNODE, :]
    dinv = dinv_ref[...]
    h = jnp.maximum(s * dinv + b_ref[...], 0.0)
    z = jnp.dot(h, w_ref[...], preferred_element_type=jnp.float32)
    z_ref[...] = z * dinv


def _tc_final_body(sp_ref, dinv_ref, b_ref, batch_ref, wl_ref, bl_ref, o_ref):
    s = sp_ref[0, :NNODE, :] + sp_ref[1, :NNODE, :]
    h = jnp.maximum(s * dinv_ref[...] + b_ref[...], 0.0)
    gi = lax.broadcasted_iota(jnp.int32, (NGRAPH, NNODE), 0)
    m = (batch_ref[...] == gi).astype(jnp.float32)
    psum = jnp.dot(m, h, preferred_element_type=jnp.float32)
    cnt = jnp.dot(m, jnp.ones((NNODE, 1), jnp.float32),
                  preferred_element_type=jnp.float32)
    pooled = psum / jnp.maximum(cnt, 1.0)
    o_ref[...] = jnp.dot(pooled, wl_ref[...],
                         preferred_element_type=jnp.float32) + bl_ref[...]


_tc_first = pl.pallas_call(
    _tc_first_body,
    out_shape=[jax.ShapeDtypeStruct((NNODE, DIM), jnp.float32),
               jax.ShapeDtypeStruct((NNODE, 1), jnp.float32)],
)

_tc_mid = pl.pallas_call(
    _tc_mid_body,
    out_shape=jax.ShapeDtypeStruct((NNODE, DIM), jnp.float32),
)

_tc_final = pl.pallas_call(
    _tc_final_body,
    out_shape=jax.ShapeDtypeStruct((NGRAPH, NCLS), jnp.float32),
)


def kernel(x, edge_index, batch, W1, b1, W2, b2, W3, b3, Wl, bl):
    loop = jnp.arange(NNODE, dtype=jnp.int32)
    pad = EPAD - ETOT
    pad_src = (jnp.arange(pad, dtype=jnp.int32) * 97) % NNODE
    pad_dst = NNODE + (jnp.arange(pad, dtype=jnp.int32) % (NPAD - NNODE))
    src = jnp.concatenate([edge_index[0], loop, pad_src])
    dst = jnp.concatenate([edge_index[1], loop, pad_dst])
    src_t = src.reshape(NWORK, NBLK, BLK)
    dst_t = dst.reshape(NWORK, NBLK, BLK)

    deg_parts = _sc_degree(dst_t)
    z, dinv = _tc_first(x, W1, deg_parts)
    s1 = _sc_propagate(z, src_t, dst_t)
    z = _tc_mid(s1, dinv, b1.reshape(1, DIM), W2)
    s2 = _sc_propagate(z, src_t, dst_t)
    z = _tc_mid(s2, dinv, b2.reshape(1, DIM), W3)
    s3 = _sc_propagate(z, src_t, dst_t)
    return _tc_final(s3, dinv, b3.reshape(1, DIM), batch.reshape(1, NNODE),
                     Wl, bl.reshape(1, NCLS))


# speedup vs baseline: 1.0505x; 1.0505x over previous
import functools

import jax
import jax.numpy as jnp
from jax import lax
from jax.experimental import pallas as pl
from jax.experimental.pallas import tpu as pltpu
from jax.experimental.pallas import tpu_sc as plsc

NNODE = 10000
NEDGE = 320000
DIM = 128
NGRAPH = 64
NCLS = 2

NCORE = 2
NSUB = 16
NWORK = NCORE * NSUB
BLK = 64
ETOT = NEDGE
CHKB = 16
NBLK = -(-(-(-ETOT // (NWORK * BLK))) // CHKB) * CHKB
NCHK = NBLK // CHKB
EPAD = NWORK * NBLK * BLK
NPAD = 10112
ROWS_PER_TILE = NPAD // NSUB
DEGW = 16
DEPTH = 4
AHEAD = 2

_mesh = plsc.VectorSubcoreMesh(core_axis_name="c", subcore_axis_name="s")


@functools.partial(
    pl.kernel,
    out_type=jax.ShapeDtypeStruct((NCORE, NPAD, DEGW), jnp.float32),
    mesh=_mesh,
    scratch_types=[
        pltpu.VMEM((CHKB, BLK), jnp.int32),
        pltpu.VMEM((BLK, DEGW), jnp.float32),
        pltpu.VMEM((8, DEGW), jnp.float32),
        pltpu.VMEM_SHARED((NPAD, DEGW), jnp.float32),
    ],
)
def _sc_degree(dst_hbm, out_hbm, didx, ones_buf, zbuf, accum):
    cid = lax.axis_index("c")
    sid = lax.axis_index("s")
    wid = cid * NSUB + sid

    @pl.loop(0, BLK)
    def _(i):
        ones_buf[i, pl.ds(0, DEGW)] = jnp.ones((DEGW,), jnp.float32)

    @pl.loop(0, 8)
    def _(i):
        zbuf[i, pl.ds(0, DEGW)] = jnp.zeros((DEGW,), jnp.float32)

    base = sid * ROWS_PER_TILE

    @pl.loop(0, ROWS_PER_TILE // 8)
    def _(r):
        pltpu.sync_copy(zbuf, accum.at[pl.ds(base + r * 8, 8)])

    plsc.subcore_barrier()

    @pl.loop(0, NCHK)
    def _(c):
        pltpu.sync_copy(dst_hbm.at[wid, pl.ds(c * CHKB, CHKB)], didx)

        @pl.loop(0, CHKB)
        def _(j):
            pltpu.sync_copy(ones_buf, accum.at[didx.at[j]], add=True)

    plsc.subcore_barrier()
    pltpu.sync_copy(accum.at[pl.ds(base, ROWS_PER_TILE)],
                    out_hbm.at[cid, pl.ds(base, ROWS_PER_TILE)])


@functools.partial(
    pl.kernel,
    out_type=jax.ShapeDtypeStruct((NCORE, NPAD, DIM), jnp.float32),
    mesh=_mesh,
    scratch_types=[
        pltpu.VMEM((CHKB, BLK), jnp.int32),
        pltpu.VMEM((CHKB, BLK), jnp.int32),
        pltpu.VMEM((CHKB, BLK), jnp.int32),
        pltpu.VMEM((CHKB, BLK), jnp.int32),
        pltpu.VMEM((BLK, DIM), jnp.float32),
        pltpu.VMEM((BLK, DIM), jnp.float32),
        pltpu.VMEM((BLK, DIM), jnp.float32),
        pltpu.VMEM((BLK, DIM), jnp.float32),
        pltpu.VMEM_SHARED((NPAD, DIM), jnp.float32),
        pltpu.SemaphoreType.DMA,
        pltpu.SemaphoreType.DMA,
        pltpu.SemaphoreType.DMA,
        pltpu.SemaphoreType.DMA,
        pltpu.SemaphoreType.DMA,
        pltpu.SemaphoreType.DMA,
        pltpu.SemaphoreType.DMA,
        pltpu.SemaphoreType.DMA,
        pltpu.SemaphoreType.DMA,
        pltpu.SemaphoreType.DMA,
    ],
)
def _sc_propagate(z_hbm, src_hbm, dst_hbm, out_hbm,
                  sidxA, didxA, sidxB, didxB,
                  buf0, buf1, buf2, buf3, accum,
                  gs0, gs1, gs2, gs3, ss0, ss1, ss2, ss3, is0, is1):
    cid = lax.axis_index("c")
    sid = lax.axis_index("s")
    wid = cid * NSUB + sid
    bufs = (buf0, buf1, buf2, buf3)
    gsem = (gs0, gs1, gs2, gs3)
    ssem = (ss0, ss1, ss2, ss3)

    @pl.loop(0, 8)
    def _(i):
        for k in range(DIM // 16):
            buf0[i, pl.ds(k * 16, 16)] = jnp.zeros((16,), jnp.float32)

    base = sid * ROWS_PER_TILE

    @pl.loop(0, ROWS_PER_TILE // 8)
    def _(r):
        pltpu.sync_copy(buf0.at[pl.ds(0, 8)], accum.at[pl.ds(base + r * 8, 8)])

    plsc.subcore_barrier()

    def chunk_body(c, sidx, didx, sidx_nxt, didx_nxt):
        @pl.when(c == 0)
        def _():
            pltpu.sync_copy(src_hbm.at[wid, pl.ds(0, CHKB)], sidx)
            pltpu.sync_copy(dst_hbm.at[wid, pl.ds(0, CHKB)], didx)

        @pl.when(c > 0)
        def _():
            pltpu.make_async_copy(
                src_hbm.at[wid, pl.ds(c * CHKB, CHKB)], sidx, is0).wait()
            pltpu.make_async_copy(
                dst_hbm.at[wid, pl.ds(c * CHKB, CHKB)], didx, is1).wait()
            for b in range(DEPTH):
                pltpu.make_async_copy(
                    bufs[b], accum.at[didx.at[0]], ssem[b]).wait()

        for jg in range(AHEAD):
            pltpu.async_copy(z_hbm.at[sidx.at[jg]], bufs[jg], gsem[jg])

        @pl.when(c + 1 < NCHK)
        def _():
            pltpu.async_copy(
                src_hbm.at[wid, pl.ds((c + 1) * CHKB, CHKB)], sidx_nxt, is0)
            pltpu.async_copy(
                dst_hbm.at[wid, pl.ds((c + 1) * CHKB, CHKB)], didx_nxt, is1)

        @pl.loop(0, CHKB, step=DEPTH)
        def _(j0):
            for b in range(DEPTH):
                j = j0 + b
                jg = j + AHEAD
                bg = (b + AHEAD) % DEPTH

                @pl.when(jg < CHKB)
                def _():
                    @pl.when(jg >= DEPTH)
                    def _():
                        pltpu.make_async_copy(
                            bufs[bg], accum.at[didx.at[0]], ssem[bg]).wait()

                    pltpu.async_copy(z_hbm.at[sidx.at[jg]], bufs[bg],
                                     gsem[bg])

                pltpu.make_async_copy(z_hbm.at[sidx.at[j]], bufs[b],
                                      gsem[b]).wait()
                pltpu.async_copy(bufs[b], accum.at[didx.at[j]], ssem[b],
                                 add=True)

    @pl.loop(0, NCHK // 2)
    def _(p):
        chunk_body(2 * p, sidxA, didxA, sidxB, didxB)
        chunk_body(2 * p + 1, sidxB, didxB, sidxA, didxA)

    for b in range(DEPTH):
        pltpu.make_async_copy(bufs[b], accum.at[didxB.at[CHKB - DEPTH + b]],
                              ssem[b]).wait()

    plsc.subcore_barrier()
    pltpu.sync_copy(accum.at[pl.ds(base, ROWS_PER_TILE)],
                    out_hbm.at[cid, pl.ds(base, ROWS_PER_TILE)])


def _tc_first_body(x_ref, w_ref, degp_ref, z_ref, dinv_ref):
    degp = degp_ref[...]
    deg = degp[0, :NNODE, 0:1] + degp[1, :NNODE, 0:1] + 1.0
    dinv = lax.rsqrt(deg)
    z = jnp.dot(x_ref[...], w_ref[...], preferred_element_type=jnp.float32)
    z_ref[...] = z * dinv
    dinv_ref[...] = dinv


def _tc_mid_body(sp_ref, z_in_ref, dinv_ref, b_ref, w_ref, z_ref):
    s = sp_ref[0, :NNODE, :] + sp_ref[1, :NNODE, :] + z_in_ref[...]
    dinv = dinv_ref[...]
    h = jnp.maximum(s * dinv + b_ref[...], 0.0)
    z = jnp.dot(h, w_ref[...], preferred_element_type=jnp.float32)
    z_ref[...] = z * dinv


def _tc_final_body(sp_ref, z_in_ref, dinv_ref, b_ref, batch_ref, wl_ref,
                   bl_ref, o_ref):
    s = sp_ref[0, :NNODE, :] + sp_ref[1, :NNODE, :] + z_in_ref[...]
    h = jnp.maximum(s * dinv_ref[...] + b_ref[...], 0.0)
    gi = lax.broadcasted_iota(jnp.int32, (NGRAPH, NNODE), 0)
    m = (batch_ref[...] == gi).astype(jnp.float32)
    psum = jnp.dot(m, h, preferred_element_type=jnp.float32)
    cnt = jnp.dot(m, jnp.ones((NNODE, 1), jnp.float32),
                  preferred_element_type=jnp.float32)
    pooled = psum / jnp.maximum(cnt, 1.0)
    o_ref[...] = jnp.dot(pooled, wl_ref[...],
                         preferred_element_type=jnp.float32) + bl_ref[...]


_tc_first = pl.pallas_call(
    _tc_first_body,
    out_shape=[jax.ShapeDtypeStruct((NNODE, DIM), jnp.float32),
               jax.ShapeDtypeStruct((NNODE, 1), jnp.float32)],
)

_tc_mid = pl.pallas_call(
    _tc_mid_body,
    out_shape=jax.ShapeDtypeStruct((NNODE, DIM), jnp.float32),
)

_tc_final = pl.pallas_call(
    _tc_final_body,
    out_shape=jax.ShapeDtypeStruct((NGRAPH, NCLS), jnp.float32),
)


def kernel(x, edge_index, batch, W1, b1, W2, b2, W3, b3, Wl, bl):
    pad = EPAD - ETOT
    pad_src = (jnp.arange(pad, dtype=jnp.int32) * 97) % NNODE
    pad_dst = NNODE + (jnp.arange(pad, dtype=jnp.int32) % (NPAD - NNODE))
    src = jnp.concatenate([edge_index[0], pad_src])
    dst = jnp.concatenate([edge_index[1], pad_dst])
    src_t = src.reshape(NWORK, NBLK, BLK)
    dst_t = dst.reshape(NWORK, NBLK, BLK)

    deg_parts = _sc_degree(dst_t)
    z, dinv = _tc_first(x, W1, deg_parts)
    s1 = _sc_propagate(z, src_t, dst_t)
    z = _tc_mid(s1, z, dinv, b1.reshape(1, DIM), W2)
    s2 = _sc_propagate(z, src_t, dst_t)
    z = _tc_mid(s2, z, dinv, b2.reshape(1, DIM), W3)
    s3 = _sc_propagate(z, src_t, dst_t)
    return _tc_final(s3, z, dinv, b3.reshape(1, DIM), batch.reshape(1, NNODE),
                     Wl, bl.reshape(1, NCLS))


# speedup vs baseline: 1.0600x; 1.0090x over previous
import functools

import jax
import jax.numpy as jnp
from jax import lax
from jax.experimental import pallas as pl
from jax.experimental.pallas import tpu as pltpu
from jax.experimental.pallas import tpu_sc as plsc

NNODE = 10000
NEDGE = 320000
DIM = 128
NGRAPH = 64
NCLS = 2

NCORE = 2
NSUB = 16
NWORK = NCORE * NSUB
BLK = 64
ETOT = NEDGE
CHKB = 16
NBLK = -(-(-(-ETOT // (NWORK * BLK))) // CHKB) * CHKB
NCHK = NBLK // CHKB
EPAD = NWORK * NBLK * BLK
NPAD = 10112
ROWS_PER_TILE = NPAD // NSUB
DEGW = 16
DEPTH = 4
AHEAD = 2

_mesh = plsc.VectorSubcoreMesh(core_axis_name="c", subcore_axis_name="s")


@functools.partial(
    pl.kernel,
    out_type=jax.ShapeDtypeStruct((NCORE, NPAD, DEGW), jnp.float32),
    mesh=_mesh,
    scratch_types=[
        pltpu.VMEM((CHKB, BLK), jnp.int32),
        pltpu.VMEM((BLK, DEGW), jnp.float32),
        pltpu.VMEM((8, DEGW), jnp.float32),
        pltpu.VMEM_SHARED((NPAD, DEGW), jnp.float32),
    ],
)
def _sc_degree(dst_hbm, out_hbm, didx, ones_buf, zbuf, accum):
    cid = lax.axis_index("c")
    sid = lax.axis_index("s")
    wid = cid * NSUB + sid

    @pl.loop(0, BLK)
    def _(i):
        ones_buf[i, pl.ds(0, DEGW)] = jnp.ones((DEGW,), jnp.float32)

    @pl.loop(0, 8)
    def _(i):
        zbuf[i, pl.ds(0, DEGW)] = jnp.zeros((DEGW,), jnp.float32)

    base = sid * ROWS_PER_TILE

    @pl.loop(0, ROWS_PER_TILE // 8)
    def _(r):
        pltpu.sync_copy(zbuf, accum.at[pl.ds(base + r * 8, 8)])

    plsc.subcore_barrier()

    @pl.loop(0, NCHK)
    def _(c):
        pltpu.sync_copy(dst_hbm.at[wid, pl.ds(c * CHKB, CHKB)], didx)

        @pl.loop(0, CHKB)
        def _(j):
            pltpu.sync_copy(ones_buf, accum.at[didx.at[j]], add=True)

    plsc.subcore_barrier()
    pltpu.sync_copy(accum.at[pl.ds(base, ROWS_PER_TILE)],
                    out_hbm.at[cid, pl.ds(base, ROWS_PER_TILE)])


@functools.partial(
    pl.kernel,
    out_type=jax.ShapeDtypeStruct((NCORE, NPAD, DIM), jnp.float32),
    mesh=_mesh,
    scratch_types=[
        pltpu.VMEM((CHKB, BLK), jnp.int32),
        pltpu.VMEM((CHKB, BLK), jnp.int32),
        pltpu.VMEM((CHKB, BLK), jnp.int32),
        pltpu.VMEM((CHKB, BLK), jnp.int32),
        pltpu.VMEM((BLK, DIM), jnp.float32),
        pltpu.VMEM((BLK, DIM), jnp.float32),
        pltpu.VMEM((BLK, DIM), jnp.float32),
        pltpu.VMEM((BLK, DIM), jnp.float32),
        pltpu.VMEM_SHARED((NPAD, DIM), jnp.float32),
        pltpu.SemaphoreType.DMA,
        pltpu.SemaphoreType.DMA,
        pltpu.SemaphoreType.DMA,
        pltpu.SemaphoreType.DMA,
        pltpu.SemaphoreType.DMA,
        pltpu.SemaphoreType.DMA,
        pltpu.SemaphoreType.DMA,
        pltpu.SemaphoreType.DMA,
        pltpu.SemaphoreType.DMA,
        pltpu.SemaphoreType.DMA,
    ],
)
def _sc_propagate(z_hbm, src_hbm, dst_hbm, out_hbm,
                  sidxA, didxA, sidxB, didxB,
                  buf0, buf1, buf2, buf3, accum,
                  gs0, gs1, gs2, gs3, ss0, ss1, ss2, ss3, is0, is1):
    cid = lax.axis_index("c")
    sid = lax.axis_index("s")
    wid = cid * NSUB + sid
    bufs = (buf0, buf1, buf2, buf3)
    gsem = (gs0, gs1, gs2, gs3)
    ssem = (ss0, ss1, ss2, ss3)

    @pl.loop(0, 8)
    def _(i):
        for k in range(DIM // 16):
            buf0[i, pl.ds(k * 16, 16)] = jnp.zeros((16,), jnp.float32)

    base = sid * ROWS_PER_TILE

    @pl.loop(0, ROWS_PER_TILE // 8)
    def _(r):
        pltpu.sync_copy(buf0.at[pl.ds(0, 8)], accum.at[pl.ds(base + r * 8, 8)])

    plsc.subcore_barrier()

    def chunk_body(c, sidx, didx, sidx_nxt, didx_nxt):
        @pl.when(c == 0)
        def _():
            pltpu.sync_copy(src_hbm.at[wid, pl.ds(0, CHKB)], sidx)
            pltpu.sync_copy(dst_hbm.at[wid, pl.ds(0, CHKB)], didx)

        @pl.when(c > 0)
        def _():
            pltpu.make_async_copy(
                src_hbm.at[wid, pl.ds(c * CHKB, CHKB)], sidx, is0).wait()
            pltpu.make_async_copy(
                dst_hbm.at[wid, pl.ds(c * CHKB, CHKB)], didx, is1).wait()

        for jg in range(AHEAD):
            pltpu.async_copy(z_hbm.at[sidx.at[jg]], bufs[jg], gsem[jg])

        @pl.when(c + 1 < NCHK)
        def _():
            pltpu.async_copy(
                src_hbm.at[wid, pl.ds((c + 1) * CHKB, CHKB)], sidx_nxt, is0)
            pltpu.async_copy(
                dst_hbm.at[wid, pl.ds((c + 1) * CHKB, CHKB)], didx_nxt, is1)

        @pl.loop(0, CHKB, step=DEPTH)
        def _(j0):
            for b in range(DEPTH):
                j = j0 + b
                jg = j + AHEAD
                bg = (b + AHEAD) % DEPTH

                @pl.when(jg < CHKB)
                def _():
                    pltpu.async_copy(z_hbm.at[sidx.at[jg]], bufs[bg],
                                     gsem[bg])

                pltpu.make_async_copy(z_hbm.at[sidx.at[j]], bufs[b],
                                      gsem[b]).wait()
                pltpu.sync_copy(bufs[b], accum.at[didx.at[j]], add=True)

    @pl.loop(0, NCHK // 2)
    def _(p):
        chunk_body(2 * p, sidxA, didxA, sidxB, didxB)
        chunk_body(2 * p + 1, sidxB, didxB, sidxA, didxA)

    plsc.subcore_barrier()
    pltpu.sync_copy(accum.at[pl.ds(base, ROWS_PER_TILE)],
                    out_hbm.at[cid, pl.ds(base, ROWS_PER_TILE)])


def _tc_first_body(x_ref, w_ref, degp_ref, z_ref, dinv_ref):
    degp = degp_ref[...]
    deg = degp[0, :NNODE, 0:1] + degp[1, :NNODE, 0:1] + 1.0
    dinv = lax.rsqrt(deg)
    z = jnp.dot(x_ref[...], w_ref[...], preferred_element_type=jnp.float32)
    z_ref[...] = z * dinv
    dinv_ref[...] = dinv


def _tc_mid_body(sp_ref, z_in_ref, dinv_ref, b_ref, w_ref, z_ref):
    s = sp_ref[0, :NNODE, :] + sp_ref[1, :NNODE, :] + z_in_ref[...]
    dinv = dinv_ref[...]
    h = jnp.maximum(s * dinv + b_ref[...], 0.0)
    z = jnp.dot(h, w_ref[...], preferred_element_type=jnp.float32)
    z_ref[...] = z * dinv


def _tc_final_body(sp_ref, z_in_ref, dinv_ref, b_ref, batch_ref, wl_ref,
                   bl_ref, o_ref):
    s = sp_ref[0, :NNODE, :] + sp_ref[1, :NNODE, :] + z_in_ref[...]
    h = jnp.maximum(s * dinv_ref[...] + b_ref[...], 0.0)
    gi = lax.broadcasted_iota(jnp.int32, (NGRAPH, NNODE), 0)
    m = (batch_ref[...] == gi).astype(jnp.float32)
    psum = jnp.dot(m, h, preferred_element_type=jnp.float32)
    cnt = jnp.dot(m, jnp.ones((NNODE, 1), jnp.float32),
                  preferred_element_type=jnp.float32)
    pooled = psum / jnp.maximum(cnt, 1.0)
    o_ref[...] = jnp.dot(pooled, wl_ref[...],
                         preferred_element_type=jnp.float32) + bl_ref[...]


_tc_first = pl.pallas_call(
    _tc_first_body,
    out_shape=[jax.ShapeDtypeStruct((NNODE, DIM), jnp.float32),
               jax.ShapeDtypeStruct((NNODE, 1), jnp.float32)],
)

_tc_mid = pl.pallas_call(
    _tc_mid_body,
    out_shape=jax.ShapeDtypeStruct((NNODE, DIM), jnp.float32),
)

_tc_final = pl.pallas_call(
    _tc_final_body,
    out_shape=jax.ShapeDtypeStruct((NGRAPH, NCLS), jnp.float32),
)


def kernel(x, edge_index, batch, W1, b1, W2, b2, W3, b3, Wl, bl):
    pad = EPAD - ETOT
    pad_src = (jnp.arange(pad, dtype=jnp.int32) * 97) % NNODE
    pad_dst = NNODE + (jnp.arange(pad, dtype=jnp.int32) % (NPAD - NNODE))
    src = jnp.concatenate([edge_index[0], pad_src])
    dst = jnp.concatenate([edge_index[1], pad_dst])
    src_t = src.reshape(NWORK, NBLK, BLK)
    dst_t = dst.reshape(NWORK, NBLK, BLK)

    deg_parts = _sc_degree(dst_t)
    z, dinv = _tc_first(x, W1, deg_parts)
    s1 = _sc_propagate(z, src_t, dst_t)
    z = _tc_mid(s1, z, dinv, b1.reshape(1, DIM), W2)
    s2 = _sc_propagate(z, src_t, dst_t)
    z = _tc_mid(s2, z, dinv, b2.reshape(1, DIM), W3)
    s3 = _sc_propagate(z, src_t, dst_t)
    return _tc_final(s3, z, dinv, b3.reshape(1, DIM), batch.reshape(1, NNODE),
                     Wl, bl.reshape(1, NCLS))
